# Initial kernel scaffold; baseline (speedup 1.0000x reference)
import jax
import jax.numpy as jnp
from jax.experimental import pallas as pl


def kernel(x, edge_idx, edge_weights, W1, b1, W2, b2, Wl, bl, W_ih, b_ih, W_hh, b_hh):
    raise NotImplementedError("write your pallas kernel here")


# speedup vs baseline: 2.8693x; 2.8693x over previous
import functools

import jax
import jax.numpy as jnp
from jax import lax
from jax.experimental import pallas as pl
from jax.experimental.pallas import tpu as pltpu
from jax.experimental.pallas import tpu_sc as plsc

NC = 2
NS = 16
NW = NC * NS
L = 16
CHUNK = 80
GROUPS = CHUNK // L
NEG = -3.0e38

_mesh = lambda: plsc.VectorSubcoreMesh(
    core_axis_name="c", subcore_axis_name="s", num_cores=NC, num_subcores=NS)


def _seg_combine(k16, v16, lane, is_max):
  tot = jnp.full((L,), NEG if is_max else 0.0, jnp.float32)
  notlast = lane < 0
  for t in range(L):
    tt = jnp.full((L,), t, jnp.int32)
    kt = jnp.take_along_axis(k16, tt, axis=0)
    vt = jnp.take_along_axis(v16, tt, axis=0)
    same = kt == k16
    if is_max:
      tot = jnp.maximum(tot, jnp.where(same, vt, NEG))
    else:
      tot = tot + jnp.where(same, vt, 0.0)
    notlast = notlast | (same & (lane < tt))
  return tot, ~notlast


def _dot(a, b):
  return lax.dot_general(a, b, (((1,), (0,)), ((), ())),
                         precision=lax.Precision.HIGHEST,
                         preferred_element_type=jnp.float32)


def _node_precompute(x_pad, W1s, W1d, Wl, b1p, blp, npad, d, hp, blk):
  def body(x_ref, w1s_ref, w1d_ref, wl_ref, b1_ref, bl_ref,
           as_ref, bd_ref, lin_ref):
    xb = x_ref[...]
    as_ref[...] = _dot(xb, w1s_ref[...]) + b1_ref[...]
    bd_ref[...] = _dot(xb, w1d_ref[...])
    lin_ref[...] = _dot(xb, wl_ref[...]) + bl_ref[...]

  return pl.pallas_call(
      body,
      grid=(npad // blk,),
      in_specs=[
          pl.BlockSpec((blk, d), lambda i: (i, 0)),
          pl.BlockSpec((d, hp), lambda i: (0, 0)),
          pl.BlockSpec((d, hp), lambda i: (0, 0)),
          pl.BlockSpec((d, d), lambda i: (0, 0)),
          pl.BlockSpec((1, hp), lambda i: (0, 0)),
          pl.BlockSpec((1, d), lambda i: (0, 0)),
      ],
      out_specs=[
          pl.BlockSpec((blk, hp), lambda i: (i, 0)),
          pl.BlockSpec((blk, hp), lambda i: (i, 0)),
          pl.BlockSpec((blk, d), lambda i: (i, 0)),
      ],
      out_shape=[
          jax.ShapeDtypeStruct((npad, hp), jnp.float32),
          jax.ShapeDtypeStruct((npad, hp), jnp.float32),
          jax.ShapeDtypeStruct((npad, d), jnp.float32),
      ],
  )(x_pad, W1s, W1d, Wl, b1p, blp)


def _edge_scores(As, Bd, src, dst, ews, W1e, W2p, e, npad, hp, de):
  epw = e // NW
  nch = epw // CHUNK
  hb = hp // L
  nv = npad // L

  def body(*refs):
    as_hbm, bd_hbm, src_hbm, dst_hbm = refs[:4]
    ew_hbms = refs[4:4 + de]
    w1e_hbm, w2_hbm, scores_hbm, segpart_hbm = refs[4 + de:8 + de]
    (src_v, dst_v, ew_v, as_v, bd_v, w1e_v, w2_v, seg_v, sco_v,
     sema, semb) = refs[8 + de:]

    cidx = lax.axis_index("c")
    sidx = lax.axis_index("s")
    wid = cidx * NS + sidx
    lane = lax.iota(jnp.int32, L)

    pltpu.sync_copy(w1e_hbm, w1e_v)
    pltpu.sync_copy(w2_hbm, w2_v)

    def init_fn(i, carry):
      seg_v[pl.ds(i * L, L)] = jnp.full((L,), NEG, jnp.float32)
      return carry
    lax.fori_loop(0, nv, init_fn, 0)

    def chunk_fn(c, carry):
      base = wid * epw + c * CHUNK
      pltpu.sync_copy(src_hbm.at[pl.ds(base, CHUNK)], src_v)
      pltpu.sync_copy(dst_hbm.at[pl.ds(base, CHUNK)], dst_v)
      for k in range(de):
        pltpu.sync_copy(ew_hbms[k].at[pl.ds(base, CHUNK)],
                        ew_v.at[pl.ds(k * CHUNK, CHUNK)])
      cpa = pltpu.async_copy(as_hbm.at[src_v], as_v, sema)
      cpb = pltpu.async_copy(bd_hbm.at[dst_v], bd_v, semb)
      cpa.wait()
      cpb.wait()

      def group_fn(g, gcarry):
        gb = g * L
        ewg = [ew_v[pl.ds(k * CHUNK + gb, L)] for k in range(de)]

        def edge_fn(j, sc16):
          jj = jnp.full((L,), j, jnp.int32)
          ewsplat = [jnp.take_along_axis(w, jj, axis=0) for w in ewg]
          acc = jnp.zeros((L,), jnp.float32)
          for i in range(hb):
            sl = pl.ds(i * L, L)
            u = as_v[gb + j, sl] + bd_v[gb + j, sl]
            for k in range(de):
              u = u + ewsplat[k] * w1e_v[pl.ds(k * hp + i * L, L)]
            u = jnp.maximum(u, 0.0)
            acc = acc + u * w2_v[sl]
          for dd in (1, 2, 4, 8):
            acc = acc + jnp.take_along_axis(acc, lane ^ dd, axis=0)
          return jnp.where(lane == j, acc, sc16)

        sc16 = lax.fori_loop(0, L, edge_fn, jnp.zeros((L,), jnp.float32))
        sco_v[pl.ds(c * CHUNK + gb, L)] = sc16

        k16 = src_v[pl.ds(gb, L)]
        vs, is_last = _seg_combine(k16, sc16, lane, is_max=True)
        cur = plsc.load_gather(seg_v, [k16])
        plsc.store_scatter(seg_v, [k16], jnp.maximum(cur, vs), mask=is_last)
        return gcarry

      lax.fori_loop(0, GROUPS, group_fn, 0)
      return carry

    lax.fori_loop(0, nch, chunk_fn, 0)
    pltpu.sync_copy(sco_v, scores_hbm.at[pl.ds(wid * epw, epw)])
    pltpu.sync_copy(seg_v, segpart_hbm.at[pl.ds(wid * npad, npad)])

  kern = pl.kernel(
      body,
      out_type=[
          jax.ShapeDtypeStruct((e,), jnp.float32),
          jax.ShapeDtypeStruct((NW * npad,), jnp.float32),
      ],
      mesh=_mesh(),
      compiler_params=pltpu.CompilerParams(needs_layout_passes=False, use_tc_tiling_on_sc=False),
      scratch_types=[
          pltpu.VMEM((CHUNK,), jnp.int32),
          pltpu.VMEM((CHUNK,), jnp.int32),
          pltpu.VMEM((de * CHUNK,), jnp.float32),
          pltpu.VMEM((CHUNK, hp), jnp.float32),
          pltpu.VMEM((CHUNK, hp), jnp.float32),
          pltpu.VMEM((de * hp,), jnp.float32),
          pltpu.VMEM((hp,), jnp.float32),
          pltpu.VMEM((npad,), jnp.float32),
          pltpu.VMEM((epw,), jnp.float32),
          pltpu.SemaphoreType.DMA,
          pltpu.SemaphoreType.DMA,
      ],
  )
  return kern(As, Bd, src, dst, *ews, W1e, W2p)


def _aggregate(lin, src, dst, scores, segpart, e, npad, nacc, d):
  epw = e // NW
  nch = epw // CHUNK
  nv = npad // L
  db = d // L
  rps = nacc // NS
  zr = 25

  def body(lin_hbm, src_hbm, dst_hbm, scores_hbm, segpart_hbm,
           hid_hbm, denp_hbm,
           seg_v, row_v, den_v, idxs_v, idxd_v, sc_v, rows_v, zer_v,
           hid_sh, sem):
    cidx = lax.axis_index("c")
    sidx = lax.axis_index("s")
    wid = cidx * NS + sidx
    lane = lax.iota(jnp.int32, L)

    pltpu.sync_copy(segpart_hbm.at[pl.ds(0, npad)], seg_v)

    def comb_fn(r, carry):
      pltpu.sync_copy(segpart_hbm.at[pl.ds(r * npad, npad)], row_v)

      def mx(i, c2):
        sl = pl.ds(i * L, L)
        seg_v[sl] = jnp.maximum(seg_v[sl], row_v[sl])
        return c2
      lax.fori_loop(0, nv, mx, 0)
      return carry
    lax.fori_loop(1, NW, comb_fn, 0)

    def z1(i, carry):
      den_v[pl.ds(i * L, L)] = jnp.zeros((L,), jnp.float32)
      return carry
    lax.fori_loop(0, nv, z1, 0)

    def z2(r, carry):
      for i in range(db):
        zer_v[r, pl.ds(i * L, L)] = jnp.zeros((L,), jnp.float32)
      return carry
    lax.fori_loop(0, zr, z2, 0)

    def z3(b, carry):
      pltpu.sync_copy(zer_v, hid_sh.at[pl.ds(sidx * rps + b * zr, zr)])
      return carry
    lax.fori_loop(0, rps // zr, z3, 0)
    plsc.subcore_barrier()

    def chunk_fn(c, carry):
      base = wid * epw + c * CHUNK
      pltpu.sync_copy(src_hbm.at[pl.ds(base, CHUNK)], idxs_v)
      pltpu.sync_copy(dst_hbm.at[pl.ds(base, CHUNK)], idxd_v)
      pltpu.sync_copy(scores_hbm.at[pl.ds(base, CHUNK)], sc_v)
      pltpu.async_copy(lin_hbm.at[idxd_v], rows_v, sem).wait()

      def group_fn(g, gcarry):
        gb = g * L
        k16 = idxs_v[pl.ds(gb, L)]
        s16 = sc_v[pl.ds(gb, L)]
        m16 = plsc.load_gather(seg_v, [k16])
        ex16 = jnp.exp(s16 - m16)

        vs, is_last = _seg_combine(k16, ex16, lane, is_max=False)
        cur = plsc.load_gather(den_v, [k16])
        plsc.store_scatter(den_v, [k16], cur + vs, mask=is_last)

        def sc_fn(j, jcarry):
          exs = jnp.take_along_axis(ex16, jnp.full((L,), j, jnp.int32), axis=0)
          for i in range(db):
            sl = pl.ds(i * L, L)
            rows_v[gb + j, sl] = rows_v[gb + j, sl] * exs
          return jcarry
        lax.fori_loop(0, L, sc_fn, 0)
        return gcarry

      lax.fori_loop(0, GROUPS, group_fn, 0)
      pltpu.sync_copy(rows_v, hid_sh.at[idxs_v], add=True)
      return carry

    lax.fori_loop(0, nch, chunk_fn, 0)
    pltpu.sync_copy(den_v, denp_hbm.at[pl.ds(wid * npad, npad)])
    plsc.subcore_barrier()
    pltpu.sync_copy(hid_sh.at[pl.ds(sidx * rps, rps)],
                    hid_hbm.at[pl.ds(cidx * nacc + sidx * rps, rps)])

  kern = pl.kernel(
      body,
      out_type=[
          jax.ShapeDtypeStruct((NC * nacc, d), jnp.float32),
          jax.ShapeDtypeStruct((NW * npad,), jnp.float32),
      ],
      mesh=_mesh(),
      compiler_params=pltpu.CompilerParams(needs_layout_passes=False, use_tc_tiling_on_sc=False),
      scratch_types=[
          pltpu.VMEM((npad,), jnp.float32),
          pltpu.VMEM((npad,), jnp.float32),
          pltpu.VMEM((npad,), jnp.float32),
          pltpu.VMEM((CHUNK,), jnp.int32),
          pltpu.VMEM((CHUNK,), jnp.int32),
          pltpu.VMEM((CHUNK,), jnp.float32),
          pltpu.VMEM((CHUNK, d), jnp.float32),
          pltpu.VMEM((zr, d), jnp.float32),
          pltpu.VMEM_SHARED((nacc, d), jnp.float32),
          pltpu.SemaphoreType.DMA,
      ],
  )
  return kern(lin, src, dst, scores, segpart)


def _gru(x_pad, hid0, hid1, denp, W_ihT, b_ih2, W_hhT, b_hh2, npad, d, blk):
  def body(x_ref, h0_ref, h1_ref, dp_ref, wih_ref, bih_ref, whh_ref, bhh_ref,
           out_ref):
    xb = x_ref[...]
    den = jnp.sum(dp_ref[...], axis=0)[:, None]
    pos = den > 0.0
    hid = (h0_ref[...] + h1_ref[...]) / jnp.where(pos, den, 1.0)
    gi = _dot(xb, wih_ref[...]) + bih_ref[...]
    gh = _dot(hid, whh_ref[...]) + bhh_ref[...]
    r = jax.nn.sigmoid(gi[:, :d] + gh[:, :d])
    z = jax.nn.sigmoid(gi[:, d:2 * d] + gh[:, d:2 * d])
    n = jnp.tanh(gi[:, 2 * d:] + r * gh[:, 2 * d:])
    out = (1.0 - z) * n + z * hid
    out_ref[...] = jnp.where(pos, out, xb)

  return pl.pallas_call(
      body,
      grid=(npad // blk,),
      in_specs=[
          pl.BlockSpec((blk, d), lambda i: (i, 0)),
          pl.BlockSpec((blk, d), lambda i: (i, 0)),
          pl.BlockSpec((blk, d), lambda i: (i, 0)),
          pl.BlockSpec((NW, blk), lambda i: (0, i)),
          pl.BlockSpec((d, 3 * d), lambda i: (0, 0)),
          pl.BlockSpec((1, 3 * d), lambda i: (0, 0)),
          pl.BlockSpec((d, 3 * d), lambda i: (0, 0)),
          pl.BlockSpec((1, 3 * d), lambda i: (0, 0)),
      ],
      out_specs=pl.BlockSpec((blk, d), lambda i: (i, 0)),
      out_shape=jax.ShapeDtypeStruct((npad, d), jnp.float32),
  )(x_pad, hid0, hid1, denp, W_ihT, b_ih2, W_hhT, b_hh2)


def kernel(x, edge_idx, edge_weights, W1, b1, W2, b2, Wl, bl,
           W_ih, b_ih, W_hh, b_hh):
  del b2
  n, d = x.shape
  e = edge_idx.shape[1]
  de = edge_weights.shape[1]
  h = 2 * d + de
  hp = ((h + L - 1) // L) * L
  blk = 640
  npad = ((n + NS * blk - 1) // (NS * blk)) * (NS * blk)

  f32 = jnp.float32
  x_pad = jnp.zeros((npad, d), f32).at[:n].set(x)
  W1s = jnp.pad(W1[:d], ((0, 0), (0, hp - h)))
  W1e = jnp.pad(W1[d:d + de], ((0, 0), (0, hp - h)))
  W1d = jnp.pad(W1[d + de:], ((0, 0), (0, hp - h)))
  W2p = jnp.pad(W2[:, 0], (0, hp - h))
  b1p = jnp.pad(b1, (0, hp - h)).reshape(1, hp)
  blp = bl.reshape(1, d)
  src = edge_idx[0]
  dst = edge_idx[1]
  ews = [edge_weights[:, k] for k in range(de)]

  As, Bd, lin = _node_precompute(x_pad, W1s, W1d, Wl, b1p, blp,
                                 npad, d, hp, blk)
  scores, segpart = _edge_scores(As, Bd, src, dst, ews, W1e.reshape(-1), W2p,
                                 e, npad, hp, de)
  nacc = ((n + NS - 1) // NS) * NS
  hidflat, denp = _aggregate(lin, src, dst, scores, segpart, e, npad, nacc, d)
  pad_rows = ((0, npad - nacc), (0, 0))
  hid0 = jnp.pad(hidflat[:nacc], pad_rows)
  hid1 = jnp.pad(hidflat[nacc:], pad_rows)
  out = _gru(x_pad, hid0, hid1, denp.reshape(NW, npad),
             W_ih.T, b_ih.reshape(1, -1), W_hh.T, b_hh.reshape(1, -1),
             npad, d, blk)
  return out[:n]


# speedup vs baseline: 3.4159x; 1.1905x over previous
import functools

import jax
import jax.numpy as jnp
from jax import lax
from jax.experimental import pallas as pl
from jax.experimental.pallas import tpu as pltpu
from jax.experimental.pallas import tpu_sc as plsc

NC = 2
NS = 16
NW = NC * NS
L = 16
CHUNK = 80
GROUPS = CHUNK // L
NEG = -3.0e38

_mesh = lambda: plsc.VectorSubcoreMesh(
    core_axis_name="c", subcore_axis_name="s", num_cores=NC, num_subcores=NS)


def _seg_combine(k16, v16, lane, is_max):
  tot = jnp.full((L,), NEG if is_max else 0.0, jnp.float32)
  notlast = lane < 0
  for t in range(L):
    tt = jnp.full((L,), t, jnp.int32)
    kt = jnp.take_along_axis(k16, tt, axis=0)
    vt = jnp.take_along_axis(v16, tt, axis=0)
    same = kt == k16
    if is_max:
      tot = jnp.maximum(tot, jnp.where(same, vt, NEG))
    else:
      tot = tot + jnp.where(same, vt, 0.0)
    notlast = notlast | (same & (lane < tt))
  return tot, ~notlast


def _dot(a, b):
  return lax.dot_general(a, b, (((1,), (0,)), ((), ())),
                         precision=lax.Precision.HIGHEST,
                         preferred_element_type=jnp.float32)


def _node_precompute(x_pad, W1s, W1d, Wl, b1p, blp, npad, d, hp, blk):
  def body(x_ref, w1s_ref, w1d_ref, wl_ref, b1_ref, bl_ref,
           as_ref, bd_ref, lin_ref):
    xb = x_ref[...]
    as_ref[...] = _dot(xb, w1s_ref[...]) + b1_ref[...]
    bd_ref[...] = _dot(xb, w1d_ref[...])
    lin_ref[...] = _dot(xb, wl_ref[...]) + bl_ref[...]

  return pl.pallas_call(
      body,
      grid=(npad // blk,),
      in_specs=[
          pl.BlockSpec((blk, d), lambda i: (i, 0)),
          pl.BlockSpec((d, hp), lambda i: (0, 0)),
          pl.BlockSpec((d, hp), lambda i: (0, 0)),
          pl.BlockSpec((d, d), lambda i: (0, 0)),
          pl.BlockSpec((1, hp), lambda i: (0, 0)),
          pl.BlockSpec((1, d), lambda i: (0, 0)),
      ],
      out_specs=[
          pl.BlockSpec((blk, hp), lambda i: (i, 0)),
          pl.BlockSpec((blk, hp), lambda i: (i, 0)),
          pl.BlockSpec((blk, d), lambda i: (i, 0)),
      ],
      out_shape=[
          jax.ShapeDtypeStruct((npad, hp), jnp.float32),
          jax.ShapeDtypeStruct((npad, hp), jnp.float32),
          jax.ShapeDtypeStruct((npad, d), jnp.float32),
      ],
  )(x_pad, W1s, W1d, Wl, b1p, blp)


def _edge_ce(edge_weights, W1e, e, de, hp, eb):
  def body(ew_ref, w1e_ref, ce_ref):
    ce_ref[...] = _dot(ew_ref[...], w1e_ref[...])

  return pl.pallas_call(
      body,
      grid=(e // eb,),
      in_specs=[
          pl.BlockSpec((eb, de), lambda i: (i, 0)),
          pl.BlockSpec((de, hp), lambda i: (0, 0)),
      ],
      out_specs=pl.BlockSpec((eb, hp), lambda i: (i, 0)),
      out_shape=jax.ShapeDtypeStruct((e, hp), jnp.float32),
  )(edge_weights, W1e)


def _edge_scores(As, Bd, Ce, src, dst, W2p, e, npad, hp):
  epw = e // NW
  nch = epw // CHUNK
  hb = hp // L
  nv = npad // L

  def body(as_hbm, bd_hbm, ce_hbm, src_hbm, dst_hbm, w2_hbm,
           scores_hbm, segpart_hbm,
           src_v, dst_v, as_v, bd_v, ce_v, w2_v, seg_v, sco_v, sema, semb):

    cidx = lax.axis_index("c")
    sidx = lax.axis_index("s")
    wid = cidx * NS + sidx
    lane = lax.iota(jnp.int32, L)

    pltpu.sync_copy(w2_hbm, w2_v)

    def init_fn(i, carry):
      seg_v[pl.ds(i * L, L)] = jnp.full((L,), NEG, jnp.float32)
      return carry
    lax.fori_loop(0, nv, init_fn, 0)

    def chunk_fn(c, carry):
      base = wid * epw + c * CHUNK
      pltpu.sync_copy(src_hbm.at[pl.ds(base, CHUNK)], src_v)
      pltpu.sync_copy(dst_hbm.at[pl.ds(base, CHUNK)], dst_v)
      pltpu.sync_copy(ce_hbm.at[pl.ds(base, CHUNK)], ce_v)
      cpa = pltpu.async_copy(as_hbm.at[src_v], as_v, sema)
      cpb = pltpu.async_copy(bd_hbm.at[dst_v], bd_v, semb)
      cpa.wait()
      cpb.wait()

      def group_fn(g, gcarry):
        gb = g * L

        def dim_fn(i, accs):
          sl = pl.ds(i * L, L)
          w2c = w2_v[sl]
          return tuple(
              accs[j] + jnp.maximum(
                  as_v[gb + j, sl] + bd_v[gb + j, sl] + ce_v[gb + j, sl],
                  0.0) * w2c
              for j in range(L))

        accs = lax.fori_loop(
            0, hb, dim_fn,
            tuple(jnp.zeros((L,), jnp.float32) for _ in range(L)))
        sc16 = jnp.zeros((L,), jnp.float32)
        for j in range(L):
          a = accs[j]
          for dd in (1, 2, 4, 8):
            a = a + jnp.take_along_axis(a, lane ^ dd, axis=0)
          sc16 = jnp.where(lane == j, a, sc16)
        sco_v[pl.ds(c * CHUNK + gb, L)] = sc16

        k16 = src_v[pl.ds(gb, L)]
        vs, is_last = _seg_combine(k16, sc16, lane, is_max=True)
        cur = plsc.load_gather(seg_v, [k16])
        plsc.store_scatter(seg_v, [k16], jnp.maximum(cur, vs), mask=is_last)
        return gcarry

      lax.fori_loop(0, GROUPS, group_fn, 0)
      return carry

    lax.fori_loop(0, nch, chunk_fn, 0)
    pltpu.sync_copy(sco_v, scores_hbm.at[pl.ds(wid * epw, epw)])
    pltpu.sync_copy(seg_v, segpart_hbm.at[pl.ds(wid * npad, npad)])

  kern = pl.kernel(
      body,
      out_type=[
          jax.ShapeDtypeStruct((e,), jnp.float32),
          jax.ShapeDtypeStruct((NW * npad,), jnp.float32),
      ],
      mesh=_mesh(),
      compiler_params=pltpu.CompilerParams(needs_layout_passes=False, use_tc_tiling_on_sc=False),
      scratch_types=[
          pltpu.VMEM((CHUNK,), jnp.int32),
          pltpu.VMEM((CHUNK,), jnp.int32),
          pltpu.VMEM((CHUNK, hp), jnp.float32),
          pltpu.VMEM((CHUNK, hp), jnp.float32),
          pltpu.VMEM((CHUNK, hp), jnp.float32),
          pltpu.VMEM((hp,), jnp.float32),
          pltpu.VMEM((npad,), jnp.float32),
          pltpu.VMEM((epw,), jnp.float32),
          pltpu.SemaphoreType.DMA,
          pltpu.SemaphoreType.DMA,
      ],
  )
  return kern(As, Bd, Ce, src, dst, W2p)


def _aggregate(lin, src, dst, scores, segpart, e, npad, nacc, d):
  epw = e // NW
  nch = epw // CHUNK
  nv = npad // L
  db = d // L
  rps = nacc // NS
  zr = 25

  def body(lin_hbm, src_hbm, dst_hbm, scores_hbm, segpart_hbm,
           hid_hbm, denp_hbm,
           seg_v, row_v, den_v, idxs_v, idxd_v, sc_v, rows_v, zer_v,
           hid_sh, sem):
    cidx = lax.axis_index("c")
    sidx = lax.axis_index("s")
    wid = cidx * NS + sidx
    lane = lax.iota(jnp.int32, L)

    pltpu.sync_copy(segpart_hbm.at[pl.ds(0, npad)], seg_v)

    def comb_fn(r, carry):
      pltpu.sync_copy(segpart_hbm.at[pl.ds(r * npad, npad)], row_v)

      def mx(i, c2):
        sl = pl.ds(i * L, L)
        seg_v[sl] = jnp.maximum(seg_v[sl], row_v[sl])
        return c2
      lax.fori_loop(0, nv, mx, 0)
      return carry
    lax.fori_loop(1, NW, comb_fn, 0)

    def z1(i, carry):
      den_v[pl.ds(i * L, L)] = jnp.zeros((L,), jnp.float32)
      return carry
    lax.fori_loop(0, nv, z1, 0)

    def z2(r, carry):
      for i in range(db):
        zer_v[r, pl.ds(i * L, L)] = jnp.zeros((L,), jnp.float32)
      return carry
    lax.fori_loop(0, zr, z2, 0)

    def z3(b, carry):
      pltpu.sync_copy(zer_v, hid_sh.at[pl.ds(sidx * rps + b * zr, zr)])
      return carry
    lax.fori_loop(0, rps // zr, z3, 0)
    plsc.subcore_barrier()

    def chunk_fn(c, carry):
      base = wid * epw + c * CHUNK
      pltpu.sync_copy(src_hbm.at[pl.ds(base, CHUNK)], idxs_v)
      pltpu.sync_copy(dst_hbm.at[pl.ds(base, CHUNK)], idxd_v)
      pltpu.sync_copy(scores_hbm.at[pl.ds(base, CHUNK)], sc_v)
      pltpu.async_copy(lin_hbm.at[idxd_v], rows_v, sem).wait()

      def group_fn(g, gcarry):
        gb = g * L
        k16 = idxs_v[pl.ds(gb, L)]
        s16 = sc_v[pl.ds(gb, L)]
        m16 = plsc.load_gather(seg_v, [k16])
        ex16 = jnp.exp(s16 - m16)

        vs, is_last = _seg_combine(k16, ex16, lane, is_max=False)
        cur = plsc.load_gather(den_v, [k16])
        plsc.store_scatter(den_v, [k16], cur + vs, mask=is_last)

        def sc_fn(j, jcarry):
          exs = jnp.take_along_axis(ex16, jnp.full((L,), j, jnp.int32), axis=0)
          for i in range(db):
            sl = pl.ds(i * L, L)
            rows_v[gb + j, sl] = rows_v[gb + j, sl] * exs
          return jcarry
        lax.fori_loop(0, L, sc_fn, 0)
        return gcarry

      lax.fori_loop(0, GROUPS, group_fn, 0)
      pltpu.sync_copy(rows_v, hid_sh.at[idxs_v], add=True)
      return carry

    lax.fori_loop(0, nch, chunk_fn, 0)
    pltpu.sync_copy(den_v, denp_hbm.at[pl.ds(wid * npad, npad)])
    plsc.subcore_barrier()
    pltpu.sync_copy(hid_sh.at[pl.ds(sidx * rps, rps)],
                    hid_hbm.at[pl.ds(cidx * nacc + sidx * rps, rps)])

  kern = pl.kernel(
      body,
      out_type=[
          jax.ShapeDtypeStruct((NC * nacc, d), jnp.float32),
          jax.ShapeDtypeStruct((NW * npad,), jnp.float32),
      ],
      mesh=_mesh(),
      compiler_params=pltpu.CompilerParams(needs_layout_passes=False, use_tc_tiling_on_sc=False),
      scratch_types=[
          pltpu.VMEM((npad,), jnp.float32),
          pltpu.VMEM((npad,), jnp.float32),
          pltpu.VMEM((npad,), jnp.float32),
          pltpu.VMEM((CHUNK,), jnp.int32),
          pltpu.VMEM((CHUNK,), jnp.int32),
          pltpu.VMEM((CHUNK,), jnp.float32),
          pltpu.VMEM((CHUNK, d), jnp.float32),
          pltpu.VMEM((zr, d), jnp.float32),
          pltpu.VMEM_SHARED((nacc, d), jnp.float32),
          pltpu.SemaphoreType.DMA,
      ],
  )
  return kern(lin, src, dst, scores, segpart)


def _gru(x_pad, hid0, hid1, denp, W_ihT, b_ih2, W_hhT, b_hh2, npad, d, blk):
  def body(x_ref, h0_ref, h1_ref, dp_ref, wih_ref, bih_ref, whh_ref, bhh_ref,
           out_ref):
    xb = x_ref[...]
    den = jnp.sum(dp_ref[...], axis=0)[:, None]
    pos = den > 0.0
    hid = (h0_ref[...] + h1_ref[...]) / jnp.where(pos, den, 1.0)
    gi = _dot(xb, wih_ref[...]) + bih_ref[...]
    gh = _dot(hid, whh_ref[...]) + bhh_ref[...]
    r = jax.nn.sigmoid(gi[:, :d] + gh[:, :d])
    z = jax.nn.sigmoid(gi[:, d:2 * d] + gh[:, d:2 * d])
    n = jnp.tanh(gi[:, 2 * d:] + r * gh[:, 2 * d:])
    out = (1.0 - z) * n + z * hid
    out_ref[...] = jnp.where(pos, out, xb)

  return pl.pallas_call(
      body,
      grid=(npad // blk,),
      in_specs=[
          pl.BlockSpec((blk, d), lambda i: (i, 0)),
          pl.BlockSpec((blk, d), lambda i: (i, 0)),
          pl.BlockSpec((blk, d), lambda i: (i, 0)),
          pl.BlockSpec((NW, blk), lambda i: (0, i)),
          pl.BlockSpec((d, 3 * d), lambda i: (0, 0)),
          pl.BlockSpec((1, 3 * d), lambda i: (0, 0)),
          pl.BlockSpec((d, 3 * d), lambda i: (0, 0)),
          pl.BlockSpec((1, 3 * d), lambda i: (0, 0)),
      ],
      out_specs=pl.BlockSpec((blk, d), lambda i: (i, 0)),
      out_shape=jax.ShapeDtypeStruct((npad, d), jnp.float32),
  )(x_pad, hid0, hid1, denp, W_ihT, b_ih2, W_hhT, b_hh2)


def kernel(x, edge_idx, edge_weights, W1, b1, W2, b2, Wl, bl,
           W_ih, b_ih, W_hh, b_hh):
  del b2
  n, d = x.shape
  e = edge_idx.shape[1]
  de = edge_weights.shape[1]
  h = 2 * d + de
  hp = ((h + L - 1) // L) * L
  blk = 640
  npad = ((n + NS * blk - 1) // (NS * blk)) * (NS * blk)

  f32 = jnp.float32
  x_pad = jnp.zeros((npad, d), f32).at[:n].set(x)
  W1s = jnp.pad(W1[:d], ((0, 0), (0, hp - h)))
  W1e = jnp.pad(W1[d:d + de], ((0, 0), (0, hp - h)))
  W1d = jnp.pad(W1[d + de:], ((0, 0), (0, hp - h)))
  W2p = jnp.pad(W2[:, 0], (0, hp - h))
  b1p = jnp.pad(b1, (0, hp - h)).reshape(1, hp)
  blp = bl.reshape(1, d)
  src = edge_idx[0]
  dst = edge_idx[1]

  As, Bd, lin = _node_precompute(x_pad, W1s, W1d, Wl, b1p, blp,
                                 npad, d, hp, blk)
  Ce = _edge_ce(edge_weights, W1e, e, de, hp, eb=8000)
  scores, segpart = _edge_scores(As, Bd, Ce, src, dst, W2p, e, npad, hp)
  nacc = ((n + NS - 1) // NS) * NS
  hidflat, denp = _aggregate(lin, src, dst, scores, segpart, e, npad, nacc, d)
  pad_rows = ((0, npad - nacc), (0, 0))
  hid0 = jnp.pad(hidflat[:nacc], pad_rows)
  hid1 = jnp.pad(hidflat[nacc:], pad_rows)
  out = _gru(x_pad, hid0, hid1, denp.reshape(NW, npad),
             W_ih.T, b_ih.reshape(1, -1), W_hh.T, b_hh.reshape(1, -1),
             npad, d, blk)
  return out[:n]


# speedup vs baseline: 5.0067x; 1.4657x over previous
import functools

import jax
import jax.numpy as jnp
from jax import lax
from jax.experimental import pallas as pl
from jax.experimental.pallas import tpu as pltpu
from jax.experimental.pallas import tpu_sc as plsc

NC = 2
NS = 16
NW = NC * NS
L = 16
CHUNK = 80
GROUPS = CHUNK // L
NEG = -3.0e38

_mesh = lambda: plsc.VectorSubcoreMesh(
    core_axis_name="c", subcore_axis_name="s", num_cores=NC, num_subcores=NS)


def _seg_combine(k16, v16, lane, is_max):
  tot = jnp.full((L,), NEG if is_max else 0.0, jnp.float32)
  notlast = lane < 0
  for t in range(L):
    tt = jnp.full((L,), t, jnp.int32)
    kt = jnp.take_along_axis(k16, tt, axis=0)
    vt = jnp.take_along_axis(v16, tt, axis=0)
    same = kt == k16
    if is_max:
      tot = jnp.maximum(tot, jnp.where(same, vt, NEG))
    else:
      tot = tot + jnp.where(same, vt, 0.0)
    notlast = notlast | (same & (lane < tt))
  return tot, ~notlast


def _dot(a, b):
  return lax.dot_general(a, b, (((1,), (0,)), ((), ())),
                         precision=lax.Precision.HIGHEST,
                         preferred_element_type=jnp.float32)


def _node_precompute(x_pad, W1s, W1d, Wl, b1p, blp, npad, d, hp, blk):
  def body(x_ref, w1s_ref, w1d_ref, wl_ref, b1_ref, bl_ref,
           as_ref, bd_ref, lin_ref):
    xb = x_ref[...]
    as_ref[...] = _dot(xb, w1s_ref[...]) + b1_ref[...]
    bd_ref[...] = _dot(xb, w1d_ref[...])
    lin_ref[...] = _dot(xb, wl_ref[...]) + bl_ref[...]

  return pl.pallas_call(
      body,
      grid=(npad // blk,),
      in_specs=[
          pl.BlockSpec((blk, d), lambda i: (i, 0)),
          pl.BlockSpec((d, hp), lambda i: (0, 0)),
          pl.BlockSpec((d, hp), lambda i: (0, 0)),
          pl.BlockSpec((d, d), lambda i: (0, 0)),
          pl.BlockSpec((1, hp), lambda i: (0, 0)),
          pl.BlockSpec((1, d), lambda i: (0, 0)),
      ],
      out_specs=[
          pl.BlockSpec((blk, hp), lambda i: (i, 0)),
          pl.BlockSpec((blk, hp), lambda i: (i, 0)),
          pl.BlockSpec((blk, d), lambda i: (i, 0)),
      ],
      out_shape=[
          jax.ShapeDtypeStruct((npad, hp), jnp.float32),
          jax.ShapeDtypeStruct((npad, hp), jnp.float32),
          jax.ShapeDtypeStruct((npad, d), jnp.float32),
      ],
  )(x_pad, W1s, W1d, Wl, b1p, blp)


def _edge_scores(As, Bd, src, dst, ews, W1e, W2p, e, npad, hp, de):
  epw = e // NW
  nch = epw // CHUNK
  hb = hp // L
  nv = npad // L

  def body(*refs):
    as_hbm, bd_hbm, src_hbm, dst_hbm = refs[:4]
    ew_hbms = refs[4:4 + de]
    w1e_hbm, w2_hbm, scores_hbm, segpart_hbm = refs[4 + de:8 + de]
    (src_v, dst_v, ew_v, as_v, bd_v, w1e_v, w2_v, seg_v, sco_v,
     sema, semb) = refs[8 + de:]

    cidx = lax.axis_index("c")
    sidx = lax.axis_index("s")
    wid = cidx * NS + sidx
    lane = lax.iota(jnp.int32, L)

    pltpu.sync_copy(w1e_hbm, w1e_v)
    pltpu.sync_copy(w2_hbm, w2_v)

    def init_fn(i, carry):
      seg_v[pl.ds(i * L, L)] = jnp.full((L,), NEG, jnp.float32)
      return carry
    lax.fori_loop(0, nv, init_fn, 0)

    def chunk_fn(c, carry):
      base = wid * epw + c * CHUNK
      pltpu.sync_copy(src_hbm.at[pl.ds(base, CHUNK)], src_v)
      pltpu.sync_copy(dst_hbm.at[pl.ds(base, CHUNK)], dst_v)
      for k in range(de):
        pltpu.sync_copy(ew_hbms[k].at[pl.ds(base, CHUNK)],
                        ew_v.at[pl.ds(k * CHUNK, CHUNK)])
      cpa = pltpu.async_copy(as_hbm.at[src_v], as_v, sema)
      cpb = pltpu.async_copy(bd_hbm.at[dst_v], bd_v, semb)
      cpa.wait()
      cpb.wait()

      def group_fn(g, gcarry):
        gb = g * L
        ewg = [ew_v[pl.ds(k * CHUNK + gb, L)] for k in range(de)]

        sc16 = jnp.zeros((L,), jnp.float32)
        for sg in range(L // 4):
          js = [sg * 4 + 0, sg * 4 + 1, sg * 4 + 2, sg * 4 + 3]
          splats = [[jnp.take_along_axis(ewg[k], jnp.full((L,), j, jnp.int32),
                                         axis=0)
                     for k in range(de)] for j in js]

          def dim_fn(i, accs, js=js, splats=splats):
            sl = pl.ds(i * L, L)
            w2c = w2_v[sl]
            w1ec = [w1e_v[pl.ds(k * hp + i * L, L)] for k in range(de)]
            new = []
            for jj, j in enumerate(js):
              u = as_v[gb + j, sl] + bd_v[gb + j, sl]
              for k in range(de):
                u = u + splats[jj][k] * w1ec[k]
              new.append(accs[jj] + jnp.maximum(u, 0.0) * w2c)
            return tuple(new)

          accs = lax.fori_loop(
              0, hb, dim_fn,
              tuple(jnp.zeros((L,), jnp.float32) for _ in range(4)))
          for jj, j in enumerate(js):
            a = accs[jj]
            for dd in (1, 2, 4, 8):
              a = a + jnp.take_along_axis(a, lane ^ dd, axis=0)
            sc16 = jnp.where(lane == j, a, sc16)
        sco_v[pl.ds(c * CHUNK + gb, L)] = sc16

        k16 = src_v[pl.ds(gb, L)]
        vs, is_last = _seg_combine(k16, sc16, lane, is_max=True)
        cur = plsc.load_gather(seg_v, [k16])
        plsc.store_scatter(seg_v, [k16], jnp.maximum(cur, vs), mask=is_last)
        return gcarry

      lax.fori_loop(0, GROUPS, group_fn, 0)
      return carry

    lax.fori_loop(0, nch, chunk_fn, 0)
    pltpu.sync_copy(sco_v, scores_hbm.at[pl.ds(wid * epw, epw)])
    pltpu.sync_copy(seg_v, segpart_hbm.at[pl.ds(wid * npad, npad)])

  kern = pl.kernel(
      body,
      out_type=[
          jax.ShapeDtypeStruct((e,), jnp.float32),
          jax.ShapeDtypeStruct((NW * npad,), jnp.float32),
      ],
      mesh=_mesh(),
      compiler_params=pltpu.CompilerParams(needs_layout_passes=False, use_tc_tiling_on_sc=False),
      scratch_types=[
          pltpu.VMEM((CHUNK,), jnp.int32),
          pltpu.VMEM((CHUNK,), jnp.int32),
          pltpu.VMEM((de * CHUNK,), jnp.float32),
          pltpu.VMEM((CHUNK, hp), jnp.float32),
          pltpu.VMEM((CHUNK, hp), jnp.float32),
          pltpu.VMEM((de * hp,), jnp.float32),
          pltpu.VMEM((hp,), jnp.float32),
          pltpu.VMEM((npad,), jnp.float32),
          pltpu.VMEM((epw,), jnp.float32),
          pltpu.SemaphoreType.DMA,
          pltpu.SemaphoreType.DMA,
      ],
  )
  return kern(As, Bd, src, dst, *ews, W1e, W2p)


def _aggregate(lin, src, dst, scores, segpart, e, npad, nacc, d):
  epw = e // NW
  nch = epw // CHUNK
  nv = npad // L
  db = d // L
  rps = nacc // NS
  zr = 25

  def body(lin_hbm, src_hbm, dst_hbm, scores_hbm, segpart_hbm,
           hid_hbm, denp_hbm,
           seg_v, row_v, den_v, idxs_v, idxd_v, sc_v, rows_v, zer_v,
           hid_sh, sem):
    cidx = lax.axis_index("c")
    sidx = lax.axis_index("s")
    wid = cidx * NS + sidx
    lane = lax.iota(jnp.int32, L)

    pltpu.sync_copy(segpart_hbm.at[pl.ds(0, npad)], seg_v)

    def comb_fn(r, carry):
      pltpu.sync_copy(segpart_hbm.at[pl.ds(r * npad, npad)], row_v)

      def mx(i, c2):
        sl = pl.ds(i * L, L)
        seg_v[sl] = jnp.maximum(seg_v[sl], row_v[sl])
        return c2
      lax.fori_loop(0, nv, mx, 0)
      return carry
    lax.fori_loop(1, NW, comb_fn, 0)

    def z1(i, carry):
      den_v[pl.ds(i * L, L)] = jnp.zeros((L,), jnp.float32)
      return carry
    lax.fori_loop(0, nv, z1, 0)

    def z2(r, carry):
      for i in range(db):
        zer_v[r, pl.ds(i * L, L)] = jnp.zeros((L,), jnp.float32)
      return carry
    lax.fori_loop(0, zr, z2, 0)

    def z3(b, carry):
      pltpu.sync_copy(zer_v, hid_sh.at[pl.ds(sidx * rps + b * zr, zr)])
      return carry
    lax.fori_loop(0, rps // zr, z3, 0)
    plsc.subcore_barrier()

    def chunk_fn(c, carry):
      base = wid * epw + c * CHUNK
      pltpu.sync_copy(src_hbm.at[pl.ds(base, CHUNK)], idxs_v)
      pltpu.sync_copy(dst_hbm.at[pl.ds(base, CHUNK)], idxd_v)
      pltpu.sync_copy(scores_hbm.at[pl.ds(base, CHUNK)], sc_v)
      pltpu.async_copy(lin_hbm.at[idxd_v], rows_v, sem).wait()

      def group_fn(g, gcarry):
        gb = g * L
        k16 = idxs_v[pl.ds(gb, L)]
        s16 = sc_v[pl.ds(gb, L)]
        m16 = plsc.load_gather(seg_v, [k16])
        ex16 = jnp.exp(s16 - m16)

        vs, is_last = _seg_combine(k16, ex16, lane, is_max=False)
        cur = plsc.load_gather(den_v, [k16])
        plsc.store_scatter(den_v, [k16], cur + vs, mask=is_last)

        def sc_fn(j, jcarry):
          exs = jnp.take_along_axis(ex16, jnp.full((L,), j, jnp.int32), axis=0)
          for i in range(db):
            sl = pl.ds(i * L, L)
            rows_v[gb + j, sl] = rows_v[gb + j, sl] * exs
          return jcarry
        lax.fori_loop(0, L, sc_fn, 0)
        return gcarry

      lax.fori_loop(0, GROUPS, group_fn, 0)
      pltpu.sync_copy(rows_v, hid_sh.at[idxs_v], add=True)
      return carry

    lax.fori_loop(0, nch, chunk_fn, 0)
    pltpu.sync_copy(den_v, denp_hbm.at[pl.ds(wid * npad, npad)])
    plsc.subcore_barrier()
    pltpu.sync_copy(hid_sh.at[pl.ds(sidx * rps, rps)],
                    hid_hbm.at[pl.ds(cidx * nacc + sidx * rps, rps)])

  kern = pl.kernel(
      body,
      out_type=[
          jax.ShapeDtypeStruct((NC * nacc, d), jnp.float32),
          jax.ShapeDtypeStruct((NW * npad,), jnp.float32),
      ],
      mesh=_mesh(),
      compiler_params=pltpu.CompilerParams(needs_layout_passes=False, use_tc_tiling_on_sc=False),
      scratch_types=[
          pltpu.VMEM((npad,), jnp.float32),
          pltpu.VMEM((npad,), jnp.float32),
          pltpu.VMEM((npad,), jnp.float32),
          pltpu.VMEM((CHUNK,), jnp.int32),
          pltpu.VMEM((CHUNK,), jnp.int32),
          pltpu.VMEM((CHUNK,), jnp.float32),
          pltpu.VMEM((CHUNK, d), jnp.float32),
          pltpu.VMEM((zr, d), jnp.float32),
          pltpu.VMEM_SHARED((nacc, d), jnp.float32),
          pltpu.SemaphoreType.DMA,
      ],
  )
  return kern(lin, src, dst, scores, segpart)


def _gru(x_pad, hid0, hid1, denp, W_ihT, b_ih2, W_hhT, b_hh2, npad, d, blk):
  def body(x_ref, h0_ref, h1_ref, dp_ref, wih_ref, bih_ref, whh_ref, bhh_ref,
           out_ref):
    xb = x_ref[...]
    den = jnp.sum(dp_ref[...], axis=0)[:, None]
    pos = den > 0.0
    hid = (h0_ref[...] + h1_ref[...]) / jnp.where(pos, den, 1.0)
    gi = _dot(xb, wih_ref[...]) + bih_ref[...]
    gh = _dot(hid, whh_ref[...]) + bhh_ref[...]
    r = jax.nn.sigmoid(gi[:, :d] + gh[:, :d])
    z = jax.nn.sigmoid(gi[:, d:2 * d] + gh[:, d:2 * d])
    n = jnp.tanh(gi[:, 2 * d:] + r * gh[:, 2 * d:])
    out = (1.0 - z) * n + z * hid
    out_ref[...] = jnp.where(pos, out, xb)

  return pl.pallas_call(
      body,
      grid=(npad // blk,),
      in_specs=[
          pl.BlockSpec((blk, d), lambda i: (i, 0)),
          pl.BlockSpec((blk, d), lambda i: (i, 0)),
          pl.BlockSpec((blk, d), lambda i: (i, 0)),
          pl.BlockSpec((NW, blk), lambda i: (0, i)),
          pl.BlockSpec((d, 3 * d), lambda i: (0, 0)),
          pl.BlockSpec((1, 3 * d), lambda i: (0, 0)),
          pl.BlockSpec((d, 3 * d), lambda i: (0, 0)),
          pl.BlockSpec((1, 3 * d), lambda i: (0, 0)),
      ],
      out_specs=pl.BlockSpec((blk, d), lambda i: (i, 0)),
      out_shape=jax.ShapeDtypeStruct((npad, d), jnp.float32),
  )(x_pad, hid0, hid1, denp, W_ihT, b_ih2, W_hhT, b_hh2)


def kernel(x, edge_idx, edge_weights, W1, b1, W2, b2, Wl, bl,
           W_ih, b_ih, W_hh, b_hh):
  del b2
  n, d = x.shape
  e = edge_idx.shape[1]
  de = edge_weights.shape[1]
  h = 2 * d + de
  hp = ((h + L - 1) // L) * L
  blk = 640
  npad = ((n + NS * blk - 1) // (NS * blk)) * (NS * blk)

  f32 = jnp.float32
  x_pad = jnp.zeros((npad, d), f32).at[:n].set(x)
  W1s = jnp.pad(W1[:d], ((0, 0), (0, hp - h)))
  W1e = jnp.pad(W1[d:d + de], ((0, 0), (0, hp - h)))
  W1d = jnp.pad(W1[d + de:], ((0, 0), (0, hp - h)))
  W2p = jnp.pad(W2[:, 0], (0, hp - h))
  b1p = jnp.pad(b1, (0, hp - h)).reshape(1, hp)
  blp = bl.reshape(1, d)
  src = edge_idx[0]
  dst = edge_idx[1]
  ews = [edge_weights[:, k] for k in range(de)]

  As, Bd, lin = _node_precompute(x_pad, W1s, W1d, Wl, b1p, blp,
                                 npad, d, hp, blk)
  scores, segpart = _edge_scores(As, Bd, src, dst, ews, W1e.reshape(-1), W2p,
                                 e, npad, hp, de)
  nacc = ((n + NS - 1) // NS) * NS
  hidflat, denp = _aggregate(lin, src, dst, scores, segpart, e, npad, nacc, d)
  pad_rows = ((0, npad - nacc), (0, 0))
  hid0 = jnp.pad(hidflat[:nacc], pad_rows)
  hid1 = jnp.pad(hidflat[nacc:], pad_rows)
  out = _gru(x_pad, hid0, hid1, denp.reshape(NW, npad),
             W_ih.T, b_ih.reshape(1, -1), W_hh.T, b_hh.reshape(1, -1),
             npad, d, blk)
  return out[:n]


# speedup vs baseline: 5.7314x; 1.1447x over previous
import functools

import jax
import jax.numpy as jnp
from jax import lax
from jax.experimental import pallas as pl
from jax.experimental.pallas import tpu as pltpu
from jax.experimental.pallas import tpu_sc as plsc

NC = 2
NS = 16
NW = NC * NS
L = 16
CHUNK = 80
GROUPS = CHUNK // L
NEG = -3.0e38

_mesh = lambda: plsc.VectorSubcoreMesh(
    core_axis_name="c", subcore_axis_name="s", num_cores=NC, num_subcores=NS)


def _seg_combine(k16, v16, lane, is_max):
  tot = jnp.full((L,), NEG if is_max else 0.0, jnp.float32)
  notlast = lane < 0
  for t in range(L):
    tt = jnp.full((L,), t, jnp.int32)
    kt = jnp.take_along_axis(k16, tt, axis=0)
    vt = jnp.take_along_axis(v16, tt, axis=0)
    same = kt == k16
    if is_max:
      tot = jnp.maximum(tot, jnp.where(same, vt, NEG))
    else:
      tot = tot + jnp.where(same, vt, 0.0)
    notlast = notlast | (same & (lane < tt))
  return tot, ~notlast


def _dot(a, b):
  return lax.dot_general(a, b, (((1,), (0,)), ((), ())),
                         precision=lax.Precision.HIGHEST,
                         preferred_element_type=jnp.float32)


def _node_precompute(x_pad, W1s, W1d, Wl, b1p, blp, npad, d, hp, blk):
  def body(x_ref, w1s_ref, w1d_ref, wl_ref, b1_ref, bl_ref,
           as_ref, bd_ref, lin_ref):
    xb = x_ref[...]
    as_ref[...] = _dot(xb, w1s_ref[...]) + b1_ref[...]
    bd_ref[...] = _dot(xb, w1d_ref[...])
    lin_ref[...] = _dot(xb, wl_ref[...]) + bl_ref[...]

  return pl.pallas_call(
      body,
      grid=(npad // blk,),
      in_specs=[
          pl.BlockSpec((blk, d), lambda i: (i, 0)),
          pl.BlockSpec((d, hp), lambda i: (0, 0)),
          pl.BlockSpec((d, hp), lambda i: (0, 0)),
          pl.BlockSpec((d, d), lambda i: (0, 0)),
          pl.BlockSpec((1, hp), lambda i: (0, 0)),
          pl.BlockSpec((1, d), lambda i: (0, 0)),
      ],
      out_specs=[
          pl.BlockSpec((blk, hp), lambda i: (i, 0)),
          pl.BlockSpec((blk, hp), lambda i: (i, 0)),
          pl.BlockSpec((blk, d), lambda i: (i, 0)),
      ],
      out_shape=[
          jax.ShapeDtypeStruct((npad, hp), jnp.float32),
          jax.ShapeDtypeStruct((npad, hp), jnp.float32),
          jax.ShapeDtypeStruct((npad, d), jnp.float32),
      ],
  )(x_pad, W1s, W1d, Wl, b1p, blp)


def _edge_scores(As, Bd, src, dst, ews, W1e, W2p, e, npad, hp, de):
  epw = e // NW
  nch = epw // CHUNK
  hb = hp // L
  nv = npad // L

  def body(*refs):
    as_hbm, bd_hbm, src_hbm, dst_hbm = refs[:4]
    ew_hbms = refs[4:4 + de]
    w1e_hbm, w2_hbm, scores_hbm, segpart_hbm = refs[4 + de:8 + de]
    (src_v0, src_v1, dst_v0, dst_v1, ew_v0, ew_v1, as_v0, as_v1,
     bd_v0, bd_v1, w1e_v, w2_v, seg_v, sco_v,
     sema0, sema1, semb0, semb1) = refs[8 + de:]
    bufs = ((src_v0, dst_v0, ew_v0, as_v0, bd_v0, sema0, semb0),
            (src_v1, dst_v1, ew_v1, as_v1, bd_v1, sema1, semb1))

    cidx = lax.axis_index("c")
    sidx = lax.axis_index("s")
    wid = cidx * NS + sidx
    lane = lax.iota(jnp.int32, L)

    pltpu.sync_copy(w1e_hbm, w1e_v)
    pltpu.sync_copy(w2_hbm, w2_v)

    def init_fn(i, carry):
      seg_v[pl.ds(i * L, L)] = jnp.full((L,), NEG, jnp.float32)
      return carry
    lax.fori_loop(0, nv, init_fn, 0)

    def issue(c, w):
      src_v, dst_v, ew_v, as_v, bd_v, sema, semb = bufs[w]
      base = wid * epw + c * CHUNK
      pltpu.sync_copy(src_hbm.at[pl.ds(base, CHUNK)], src_v)
      pltpu.sync_copy(dst_hbm.at[pl.ds(base, CHUNK)], dst_v)
      for k in range(de):
        pltpu.sync_copy(ew_hbms[k].at[pl.ds(base, CHUNK)],
                        ew_v.at[pl.ds(k * CHUNK, CHUNK)])
      pltpu.async_copy(as_hbm.at[src_v], as_v, sema)
      pltpu.async_copy(bd_hbm.at[dst_v], bd_v, semb)

    def drain(w):
      src_v, dst_v, ew_v, as_v, bd_v, sema, semb = bufs[w]
      pltpu.make_async_copy(as_hbm.at[pl.ds(0, CHUNK)], as_v, sema).wait()
      pltpu.make_async_copy(bd_hbm.at[pl.ds(0, CHUNK)], bd_v, semb).wait()

    def compute(c, w):
      src_v, dst_v, ew_v, as_v, bd_v, sema, semb = bufs[w]

      def group_fn(g, gcarry):
        gb = g * L
        ewg = [ew_v[pl.ds(k * CHUNK + gb, L)] for k in range(de)]

        sc16 = jnp.zeros((L,), jnp.float32)
        for sg in range(L // 4):
          js = [sg * 4 + 0, sg * 4 + 1, sg * 4 + 2, sg * 4 + 3]
          splats = [[jnp.take_along_axis(ewg[k], jnp.full((L,), j, jnp.int32),
                                         axis=0)
                     for k in range(de)] for j in js]

          def dim_fn(i, accs, js=js, splats=splats):
            sl = pl.ds(i * L, L)
            w2c = w2_v[sl]
            w1ec = [w1e_v[pl.ds(k * hp + i * L, L)] for k in range(de)]
            new = []
            for jj, j in enumerate(js):
              u = as_v[gb + j, sl] + bd_v[gb + j, sl]
              for k in range(de):
                u = u + splats[jj][k] * w1ec[k]
              new.append(accs[jj] + jnp.maximum(u, 0.0) * w2c)
            return tuple(new)

          accs = lax.fori_loop(
              0, hb, dim_fn,
              tuple(jnp.zeros((L,), jnp.float32) for _ in range(4)))
          for jj, j in enumerate(js):
            a = accs[jj]
            for dd in (1, 2, 4, 8):
              a = a + jnp.take_along_axis(a, lane ^ dd, axis=0)
            sc16 = jnp.where(lane == j, a, sc16)
        sco_v[pl.ds(c * CHUNK + gb, L)] = sc16

        k16 = src_v[pl.ds(gb, L)]
        vs, is_last = _seg_combine(k16, sc16, lane, is_max=True)
        cur = plsc.load_gather(seg_v, [k16])
        plsc.store_scatter(seg_v, [k16], jnp.maximum(cur, vs), mask=is_last)
        return gcarry

      lax.fori_loop(0, GROUPS, group_fn, 0)

    issue(0, 0)

    def pair_fn(t, carry):
      issue(2 * t + 1, 1)
      drain(0)
      compute(2 * t, 0)

      @pl.when(2 * t + 2 < nch)
      def _():
        issue(2 * t + 2, 0)

      drain(1)
      compute(2 * t + 1, 1)
      return carry
    lax.fori_loop(0, nch // 2, pair_fn, 0)
    if nch % 2 == 1:
      drain(0)
      compute(nch - 1, 0)

    pltpu.sync_copy(sco_v, scores_hbm.at[pl.ds(wid * epw, epw)])
    pltpu.sync_copy(seg_v, segpart_hbm.at[pl.ds(wid * npad, npad)])

  kern = pl.kernel(
      body,
      out_type=[
          jax.ShapeDtypeStruct((e,), jnp.float32),
          jax.ShapeDtypeStruct((NW * npad,), jnp.float32),
      ],
      mesh=_mesh(),
      compiler_params=pltpu.CompilerParams(needs_layout_passes=False, use_tc_tiling_on_sc=False),
      scratch_types=[
          pltpu.VMEM((CHUNK,), jnp.int32),
          pltpu.VMEM((CHUNK,), jnp.int32),
          pltpu.VMEM((CHUNK,), jnp.int32),
          pltpu.VMEM((CHUNK,), jnp.int32),
          pltpu.VMEM((de * CHUNK,), jnp.float32),
          pltpu.VMEM((de * CHUNK,), jnp.float32),
          pltpu.VMEM((CHUNK, hp), jnp.float32),
          pltpu.VMEM((CHUNK, hp), jnp.float32),
          pltpu.VMEM((CHUNK, hp), jnp.float32),
          pltpu.VMEM((CHUNK, hp), jnp.float32),
          pltpu.VMEM((de * hp,), jnp.float32),
          pltpu.VMEM((hp,), jnp.float32),
          pltpu.VMEM((npad,), jnp.float32),
          pltpu.VMEM((epw,), jnp.float32),
          pltpu.SemaphoreType.DMA,
          pltpu.SemaphoreType.DMA,
          pltpu.SemaphoreType.DMA,
          pltpu.SemaphoreType.DMA,
      ],
  )
  return kern(As, Bd, src, dst, *ews, W1e, W2p)


def _aggregate(lin, src, dst, scores, segpart, e, npad, nacc, d):
  epw = e // NW
  nch = epw // CHUNK
  nv = npad // L
  db = d // L
  rps = nacc // NS
  zr = 25

  def body(lin_hbm, src_hbm, dst_hbm, scores_hbm, segpart_hbm,
           hid_hbm, denp_hbm,
           seg_v, row_v, den_v, idxs_v, idxd_v, sc_v, rows_v, zer_v,
           hid_sh, sem):
    cidx = lax.axis_index("c")
    sidx = lax.axis_index("s")
    wid = cidx * NS + sidx
    lane = lax.iota(jnp.int32, L)

    pltpu.sync_copy(segpart_hbm.at[pl.ds(0, npad)], seg_v)

    def comb_fn(r, carry):
      pltpu.sync_copy(segpart_hbm.at[pl.ds(r * npad, npad)], row_v)

      def mx(i, c2):
        sl = pl.ds(i * L, L)
        seg_v[sl] = jnp.maximum(seg_v[sl], row_v[sl])
        return c2
      lax.fori_loop(0, nv, mx, 0)
      return carry
    lax.fori_loop(1, NW, comb_fn, 0)

    def z1(i, carry):
      den_v[pl.ds(i * L, L)] = jnp.zeros((L,), jnp.float32)
      return carry
    lax.fori_loop(0, nv, z1, 0)

    def z2(r, carry):
      for i in range(db):
        zer_v[r, pl.ds(i * L, L)] = jnp.zeros((L,), jnp.float32)
      return carry
    lax.fori_loop(0, zr, z2, 0)

    def z3(b, carry):
      pltpu.sync_copy(zer_v, hid_sh.at[pl.ds(sidx * rps + b * zr, zr)])
      return carry
    lax.fori_loop(0, rps // zr, z3, 0)
    plsc.subcore_barrier()

    def chunk_fn(c, carry):
      base = wid * epw + c * CHUNK
      pltpu.sync_copy(src_hbm.at[pl.ds(base, CHUNK)], idxs_v)
      pltpu.sync_copy(dst_hbm.at[pl.ds(base, CHUNK)], idxd_v)
      pltpu.sync_copy(scores_hbm.at[pl.ds(base, CHUNK)], sc_v)
      pltpu.async_copy(lin_hbm.at[idxd_v], rows_v, sem).wait()

      def group_fn(g, gcarry):
        gb = g * L
        k16 = idxs_v[pl.ds(gb, L)]
        s16 = sc_v[pl.ds(gb, L)]
        m16 = plsc.load_gather(seg_v, [k16])
        ex16 = jnp.exp(s16 - m16)

        vs, is_last = _seg_combine(k16, ex16, lane, is_max=False)
        cur = plsc.load_gather(den_v, [k16])
        plsc.store_scatter(den_v, [k16], cur + vs, mask=is_last)

        def sc_fn(j, jcarry):
          exs = jnp.take_along_axis(ex16, jnp.full((L,), j, jnp.int32), axis=0)
          for i in range(db):
            sl = pl.ds(i * L, L)
            rows_v[gb + j, sl] = rows_v[gb + j, sl] * exs
          return jcarry
        lax.fori_loop(0, L, sc_fn, 0)
        return gcarry

      lax.fori_loop(0, GROUPS, group_fn, 0)
      pltpu.sync_copy(rows_v, hid_sh.at[idxs_v], add=True)
      return carry

    lax.fori_loop(0, nch, chunk_fn, 0)
    pltpu.sync_copy(den_v, denp_hbm.at[pl.ds(wid * npad, npad)])
    plsc.subcore_barrier()
    pltpu.sync_copy(hid_sh.at[pl.ds(sidx * rps, rps)],
                    hid_hbm.at[pl.ds(cidx * nacc + sidx * rps, rps)])

  kern = pl.kernel(
      body,
      out_type=[
          jax.ShapeDtypeStruct((NC * nacc, d), jnp.float32),
          jax.ShapeDtypeStruct((NW * npad,), jnp.float32),
      ],
      mesh=_mesh(),
      compiler_params=pltpu.CompilerParams(needs_layout_passes=False, use_tc_tiling_on_sc=False),
      scratch_types=[
          pltpu.VMEM((npad,), jnp.float32),
          pltpu.VMEM((npad,), jnp.float32),
          pltpu.VMEM((npad,), jnp.float32),
          pltpu.VMEM((CHUNK,), jnp.int32),
          pltpu.VMEM((CHUNK,), jnp.int32),
          pltpu.VMEM((CHUNK,), jnp.float32),
          pltpu.VMEM((CHUNK, d), jnp.float32),
          pltpu.VMEM((zr, d), jnp.float32),
          pltpu.VMEM_SHARED((nacc, d), jnp.float32),
          pltpu.SemaphoreType.DMA,
      ],
  )
  return kern(lin, src, dst, scores, segpart)


def _gru(x_pad, hid0, hid1, denp, W_ihT, b_ih2, W_hhT, b_hh2, npad, d, blk):
  def body(x_ref, h0_ref, h1_ref, dp_ref, wih_ref, bih_ref, whh_ref, bhh_ref,
           out_ref):
    xb = x_ref[...]
    den = jnp.sum(dp_ref[...], axis=0)[:, None]
    pos = den > 0.0
    hid = (h0_ref[...] + h1_ref[...]) / jnp.where(pos, den, 1.0)
    gi = _dot(xb, wih_ref[...]) + bih_ref[...]
    gh = _dot(hid, whh_ref[...]) + bhh_ref[...]
    r = jax.nn.sigmoid(gi[:, :d] + gh[:, :d])
    z = jax.nn.sigmoid(gi[:, d:2 * d] + gh[:, d:2 * d])
    n = jnp.tanh(gi[:, 2 * d:] + r * gh[:, 2 * d:])
    out = (1.0 - z) * n + z * hid
    out_ref[...] = jnp.where(pos, out, xb)

  return pl.pallas_call(
      body,
      grid=(npad // blk,),
      in_specs=[
          pl.BlockSpec((blk, d), lambda i: (i, 0)),
          pl.BlockSpec((blk, d), lambda i: (i, 0)),
          pl.BlockSpec((blk, d), lambda i: (i, 0)),
          pl.BlockSpec((NW, blk), lambda i: (0, i)),
          pl.BlockSpec((d, 3 * d), lambda i: (0, 0)),
          pl.BlockSpec((1, 3 * d), lambda i: (0, 0)),
          pl.BlockSpec((d, 3 * d), lambda i: (0, 0)),
          pl.BlockSpec((1, 3 * d), lambda i: (0, 0)),
      ],
      out_specs=pl.BlockSpec((blk, d), lambda i: (i, 0)),
      out_shape=jax.ShapeDtypeStruct((npad, d), jnp.float32),
  )(x_pad, hid0, hid1, denp, W_ihT, b_ih2, W_hhT, b_hh2)


def kernel(x, edge_idx, edge_weights, W1, b1, W2, b2, Wl, bl,
           W_ih, b_ih, W_hh, b_hh):
  del b2
  n, d = x.shape
  e = edge_idx.shape[1]
  de = edge_weights.shape[1]
  h = 2 * d + de
  hp = ((h + L - 1) // L) * L
  blk = 640
  npad = ((n + NS * blk - 1) // (NS * blk)) * (NS * blk)

  f32 = jnp.float32
  x_pad = jnp.zeros((npad, d), f32).at[:n].set(x)
  W1s = jnp.pad(W1[:d], ((0, 0), (0, hp - h)))
  W1e = jnp.pad(W1[d:d + de], ((0, 0), (0, hp - h)))
  W1d = jnp.pad(W1[d + de:], ((0, 0), (0, hp - h)))
  W2p = jnp.pad(W2[:, 0], (0, hp - h))
  b1p = jnp.pad(b1, (0, hp - h)).reshape(1, hp)
  blp = bl.reshape(1, d)
  src = edge_idx[0]
  dst = edge_idx[1]
  ews = [edge_weights[:, k] for k in range(de)]

  As, Bd, lin = _node_precompute(x_pad, W1s, W1d, Wl, b1p, blp,
                                 npad, d, hp, blk)
  scores, segpart = _edge_scores(As, Bd, src, dst, ews, W1e.reshape(-1), W2p,
                                 e, npad, hp, de)
  nacc = ((n + NS - 1) // NS) * NS
  hidflat, denp = _aggregate(lin, src, dst, scores, segpart, e, npad, nacc, d)
  pad_rows = ((0, npad - nacc), (0, 0))
  hid0 = jnp.pad(hidflat[:nacc], pad_rows)
  hid1 = jnp.pad(hidflat[nacc:], pad_rows)
  out = _gru(x_pad, hid0, hid1, denp.reshape(NW, npad),
             W_ih.T, b_ih.reshape(1, -1), W_hh.T, b_hh.reshape(1, -1),
             npad, d, blk)
  return out[:n]
